# Initial kernel scaffold; baseline (speedup 1.0000x reference)
import jax
import jax.numpy as jnp
from jax.experimental import pallas as pl


def kernel(num_prop, cat_prop, tweet_emb, user_emb, user_name_emb, edge_index, edge_type, W_num, b_num, W_cat, b_cat, W_tweet, b_tweet, W_user, b_user, W_uname, b_uname, W_in, b_in, W_root, b_rgcn, W_rel, W_o1, b_o1, W_o2, b_o2, W_o3, b_o3):
    raise NotImplementedError("write your pallas kernel here")


# speedup vs baseline: 1.5237x; 1.5237x over previous
import functools

import jax
import jax.numpy as jnp
from jax import lax
from jax.experimental import pallas as pl
from jax.experimental.pallas import tpu as pltpu

N = 50000
E = 800000
DIM = 100
IND = 20


def _leaky(x):
    return jnp.where(x > 0, x, 0.01 * x)


def _enc_body(num_ref, cat_ref, tw_ref, us_ref, un_ref,
              Wn_ref, bn_ref, Wc_ref, bc_ref, Wt_ref, bt_ref,
              Wu_ref, bu_ref, Wun_ref, bun_ref, Win_ref, bin_ref,
              out_ref):
    n = _leaky(num_ref[...] @ Wn_ref[...] + bn_ref[...])
    c = _leaky(cat_ref[...] @ Wc_ref[...] + bc_ref[...])
    t = _leaky(tw_ref[...] @ Wt_ref[...] + bt_ref[...])
    u = _leaky(us_ref[...] @ Wu_ref[...] + bu_ref[...])
    un = _leaky(un_ref[...] @ Wun_ref[...] + bun_ref[...])
    Win = Win_ref[...]
    acc = n @ Win[0:IND] + c @ Win[IND:2 * IND] + t @ Win[2 * IND:3 * IND]
    acc = acc + u @ Win[3 * IND:4 * IND] + un @ Win[4 * IND:5 * IND]
    out_ref[...] = _leaky(acc + bin_ref[...])


def _encode(num_prop, cat_prop, tweet_emb, user_emb, user_name_emb,
            W_num, b_num, W_cat, b_cat, W_tweet, b_tweet, W_user, b_user,
            W_uname, b_uname, W_in, b_in):
    B = min(400, N)
    grid = (N // B,)
    row = lambda i: (i, 0)
    fixed = lambda i: (0, 0)
    fixed1 = lambda i: (0,)
    return pl.pallas_call(
        _enc_body,
        grid=grid,
        in_specs=[
            pl.BlockSpec((B, 5), row),
            pl.BlockSpec((B, 3), row),
            pl.BlockSpec((B, 768), row),
            pl.BlockSpec((B, 768), row),
            pl.BlockSpec((B, 768), row),
            pl.BlockSpec((5, IND), fixed),
            pl.BlockSpec((IND,), fixed1),
            pl.BlockSpec((3, IND), fixed),
            pl.BlockSpec((IND,), fixed1),
            pl.BlockSpec((768, IND), fixed),
            pl.BlockSpec((IND,), fixed1),
            pl.BlockSpec((768, IND), fixed),
            pl.BlockSpec((IND,), fixed1),
            pl.BlockSpec((768, IND), fixed),
            pl.BlockSpec((IND,), fixed1),
            pl.BlockSpec((5 * IND, DIM), fixed),
            pl.BlockSpec((DIM,), fixed1),
        ],
        out_specs=pl.BlockSpec((B, DIM), row),
        out_shape=jax.ShapeDtypeStruct((N, DIM), jnp.float32),
    )(num_prop, cat_prop, tweet_emb, user_emb, user_name_emb,
      W_num, b_num, W_cat, b_cat, W_tweet, b_tweet, W_user, b_user,
      W_uname, b_uname, W_in, b_in)


def _combine_body(x_ref, seg_ref, inv_ref, Wr_ref, b_ref, Wrel_ref, out_ref):
    x = x_ref[...]
    seg = seg_ref[...]
    inv = inv_ref[...]
    m0 = seg[:, :DIM] * inv[:, 0:1]
    m1 = seg[:, DIM:] * inv[:, 1:2]
    out_ref[...] = (x @ Wr_ref[...] + b_ref[...]
                    + m0 @ Wrel_ref[0] + m1 @ Wrel_ref[1])


def _combine(x, seg, inv, W_root, b, W_rel):
    B = min(2000, N)
    grid = (N // B,)
    row = lambda i: (i, 0)
    return pl.pallas_call(
        _combine_body,
        grid=grid,
        in_specs=[
            pl.BlockSpec((B, DIM), row),
            pl.BlockSpec((B, 2 * DIM), row),
            pl.BlockSpec((B, 2), row),
            pl.BlockSpec((DIM, DIM), lambda i: (0, 0)),
            pl.BlockSpec((DIM,), lambda i: (0,)),
            pl.BlockSpec((2, DIM, DIM), lambda i: (0, 0, 0)),
        ],
        out_specs=pl.BlockSpec((B, DIM), row),
        out_shape=jax.ShapeDtypeStruct((N, DIM), jnp.float32),
    )(x, seg, inv, W_root, b, W_rel)


def _head_body(x_ref, W1_ref, b1_ref, W2_ref, b2_ref, W3_ref, b3_ref, out_ref):
    h = _leaky(x_ref[...] @ W1_ref[...] + b1_ref[...])
    h = _leaky(h @ W2_ref[...] + b2_ref[...])
    out_ref[...] = h @ W3_ref[...] + b3_ref[...]


def _head(x, W1, b1, W2, b2, W3, b3):
    B = min(2000, N)
    grid = (N // B,)
    row = lambda i: (i, 0)
    fixed = lambda i: (0, 0)
    fixed1 = lambda i: (0,)
    return pl.pallas_call(
        _head_body,
        grid=grid,
        in_specs=[
            pl.BlockSpec((B, DIM), row),
            pl.BlockSpec((DIM, DIM), fixed),
            pl.BlockSpec((DIM,), fixed1),
            pl.BlockSpec((DIM, DIM), fixed),
            pl.BlockSpec((DIM,), fixed1),
            pl.BlockSpec((DIM, 2), fixed),
            pl.BlockSpec((2,), fixed1),
        ],
        out_specs=pl.BlockSpec((B, 2), row),
        out_shape=jax.ShapeDtypeStruct((N, 2), jnp.float32),
    )(x, W1, b1, W2, b2, W3, b3)


def kernel(num_prop, cat_prop, tweet_emb, user_emb, user_name_emb, edge_index,
           edge_type, W_num, b_num, W_cat, b_cat, W_tweet, b_tweet, W_user,
           b_user, W_uname, b_uname, W_in, b_in, W_root, b_rgcn, W_rel,
           W_o1, b_o1, W_o2, b_o2, W_o3, b_o3):
    src = edge_index[0].astype(jnp.int32)
    dst = edge_index[1].astype(jnp.int32)
    etype = edge_type.astype(jnp.int32)
    seg_id = dst * 2 + etype

    x = _encode(num_prop, cat_prop, tweet_emb, user_emb, user_name_emb,
                W_num, b_num, W_cat, b_cat, W_tweet, b_tweet, W_user, b_user,
                W_uname, b_uname, W_in, b_in)

    cnt = jax.ops.segment_sum(jnp.ones((E,), jnp.float32), seg_id,
                              num_segments=2 * N)
    inv = (1.0 / jnp.clip(cnt, 1.0)).reshape(N, 2)

    for _ in range(2):
        xs = jnp.take(x, src, axis=0)
        seg = jax.ops.segment_sum(xs, seg_id, num_segments=2 * N)
        seg = seg.reshape(N, 2 * DIM)
        x = _combine(x, seg, inv, W_root, b_rgcn, W_rel)

    return _head(x, W_o1, b_o1, W_o2, b_o2, W_o3, b_o3)
